# Initial kernel scaffold; baseline (speedup 1.0000x reference)
import jax
import jax.numpy as jnp
from jax.experimental import pallas as pl


def kernel(x, edge_index, graph_ids, H1_w, H1_b, W1_w, W1_b, H2_w, H2_b, W2_w, W2_b, H3_w, H3_b, W3_w, W3_b, fc_w, fc_b):
    raise NotImplementedError("write your pallas kernel here")


# speedup vs baseline: 3.3467x; 3.3467x over previous
import functools

import jax
import jax.numpy as jnp
from jax import lax
from jax.experimental import pallas as pl
from jax.experimental.pallas import tpu as pltpu
from jax.experimental.pallas import tpu_sc as plsc

N = 10000
E = 160000
D = 256
DH = 128
G = 64
NS = 16
CHUNK = 80
EPT = E // NS
NCHUNKS = EPT // CHUNK
ROWS_PT = 624
ROWS_REM = N - NS * ROWS_PT

BLK = 1000
NB = N // BLK


def _sc_agg_body(fl_hbm, fh_hbm, src_hbm, dst_hbm, ol_hbm, oh_hbm,
                 acc, sidx, didx, rows, sem):
    cid = lax.axis_index("c")
    sid = lax.axis_index("s")
    r0 = sid * ROWS_PT
    rem0 = NS * ROWS_PT

    @pl.when(cid == 0)
    def _():
        pltpu.sync_copy(fl_hbm.at[pl.ds(r0, ROWS_PT)], acc.at[pl.ds(r0, ROWS_PT)])

        @pl.when(sid == 0)
        def _():
            pltpu.sync_copy(fl_hbm.at[pl.ds(rem0, ROWS_REM)],
                            acc.at[pl.ds(rem0, ROWS_REM)])

    @pl.when(cid == 1)
    def _():
        pltpu.sync_copy(fh_hbm.at[pl.ds(r0, ROWS_PT)], acc.at[pl.ds(r0, ROWS_PT)])

        @pl.when(sid == 0)
        def _():
            pltpu.sync_copy(fh_hbm.at[pl.ds(rem0, ROWS_REM)],
                            acc.at[pl.ds(rem0, ROWS_REM)])

    plsc.subcore_barrier()

    e0 = sid * EPT

    def body(i, carry):
        base = e0 + i * CHUNK
        pltpu.sync_copy(src_hbm.at[pl.ds(base, CHUNK)], sidx)
        pltpu.sync_copy(dst_hbm.at[pl.ds(base, CHUNK)], didx)

        @pl.when(cid == 0)
        def _():
            pltpu.async_copy(fl_hbm.at[sidx], rows, sem).wait()

        @pl.when(cid == 1)
        def _():
            pltpu.async_copy(fh_hbm.at[sidx], rows, sem).wait()

        pltpu.sync_copy(rows, acc.at[didx], add=True)
        return carry

    lax.fori_loop(0, NCHUNKS, body, 0)
    plsc.subcore_barrier()

    @pl.when(cid == 0)
    def _():
        pltpu.sync_copy(acc.at[pl.ds(r0, ROWS_PT)], ol_hbm.at[pl.ds(r0, ROWS_PT)])

        @pl.when(sid == 0)
        def _():
            pltpu.sync_copy(acc.at[pl.ds(rem0, ROWS_REM)],
                            ol_hbm.at[pl.ds(rem0, ROWS_REM)])

    @pl.when(cid == 1)
    def _():
        pltpu.sync_copy(acc.at[pl.ds(r0, ROWS_PT)], oh_hbm.at[pl.ds(r0, ROWS_PT)])

        @pl.when(sid == 0)
        def _():
            pltpu.sync_copy(acc.at[pl.ds(rem0, ROWS_REM)],
                            oh_hbm.at[pl.ds(rem0, ROWS_REM)])


def _sc_aggregate(fl, fh, src, dst):
    mesh = plsc.VectorSubcoreMesh(core_axis_name="c", subcore_axis_name="s")
    f = pl.kernel(
        _sc_agg_body,
        out_type=(jax.ShapeDtypeStruct((N, DH), jnp.float32),
                  jax.ShapeDtypeStruct((N, DH), jnp.float32)),
        mesh=mesh,
        scratch_types=(
            pltpu.VMEM_SHARED((N, DH), jnp.float32),
            pltpu.VMEM((CHUNK,), jnp.int32),
            pltpu.VMEM((CHUNK,), jnp.int32),
            pltpu.VMEM((CHUNK, DH), jnp.float32),
            pltpu.SemaphoreType.DMA,
        ),
        name="sc_edge_aggregate",
    )
    return f(fl, fh, src, dst)


def _dotT(a, b):
    return lax.dot_general(a, b, (((1,), (1,)), ((), ())),
                           preferred_element_type=jnp.float32,
                           precision=lax.Precision.HIGHEST)


def _tc_level_body(gid_ref, al_ref, ah_ref, hw_ref, hb_ref, ww_ref, wb_ref,
                   nl_ref, nh_ref, fp_ref):
    step = pl.program_id(0)
    agg = jnp.concatenate([al_ref[...], ah_ref[...]], axis=1)
    nf = jnp.maximum(_dotT(agg, hw_ref[...]) + hb_ref[...], 0.0)
    logits = _dotT(nf, ww_ref[...]) + wb_ref[...]
    m = jnp.max(logits, axis=1, keepdims=True)
    ex = jnp.exp(logits - m)
    sparse = ex / jnp.sum(ex, axis=1, keepdims=True)
    nl_ref[...] = nf[:, :DH]
    nh_ref[...] = nf[:, DH:]
    gid = gid_ref[...].reshape(BLK, 1)
    onehot = (gid == lax.broadcasted_iota(jnp.int32, (BLK, G), 1)).astype(jnp.float32)
    fpb = lax.dot_general(onehot, sparse, (((0,), (0,)), ((), ())),
                          preferred_element_type=jnp.float32,
                          precision=lax.Precision.HIGHEST)

    @pl.when(step == 0)
    def _():
        fp_ref[...] = fpb

    @pl.when(step != 0)
    def _():
        fp_ref[...] += fpb


def _tc_level(gid3, al, ah, hw, hb, ww, wb):
    return pl.pallas_call(
        _tc_level_body,
        grid=(NB,),
        in_specs=[
            pl.BlockSpec((1, 1, BLK), lambda i: (i, 0, 0)),
            pl.BlockSpec((BLK, DH), lambda i: (i, 0)),
            pl.BlockSpec((BLK, DH), lambda i: (i, 0)),
            pl.BlockSpec((D, D), lambda i: (0, 0)),
            pl.BlockSpec((1, D), lambda i: (0, 0)),
            pl.BlockSpec((D, D), lambda i: (0, 0)),
            pl.BlockSpec((1, D), lambda i: (0, 0)),
        ],
        out_specs=[
            pl.BlockSpec((BLK, DH), lambda i: (i, 0)),
            pl.BlockSpec((BLK, DH), lambda i: (i, 0)),
            pl.BlockSpec((G, D), lambda i: (0, 0)),
        ],
        out_shape=[
            jax.ShapeDtypeStruct((N, DH), jnp.float32),
            jax.ShapeDtypeStruct((N, DH), jnp.float32),
            jax.ShapeDtypeStruct((G, D), jnp.float32),
        ],
        name="tc_level_dense",
    )(gid3, al, ah, hw, hb, ww, wb)


def _fc_body(fp1_ref, fp2_ref, fp3_ref, fcw_ref, fcb_ref, out_ref):
    fp = fp1_ref[...] + fp2_ref[...] + fp3_ref[...]
    prod = fp * fcw_ref[...]
    out_ref[...] = jnp.sum(prod, axis=1, keepdims=True) + fcb_ref[...]


def _fc_out(fp1, fp2, fp3, fcw, fcb):
    return pl.pallas_call(
        _fc_body,
        out_shape=jax.ShapeDtypeStruct((G, 1), jnp.float32),
        name="tc_fc_out",
    )(fp1, fp2, fp3, fcw, fcb)


def kernel(x, edge_index, graph_ids,
           H1_w, H1_b, W1_w, W1_b,
           H2_w, H2_b, W2_w, W2_b,
           H3_w, H3_b, W3_w, W3_b,
           fc_w, fc_b):
    src = edge_index[0]
    dst = edge_index[1]
    fl = x[:, :DH]
    fh = x[:, DH:]
    gid3 = graph_ids.reshape(NB, 1, BLK)
    levels = [(H1_w, H1_b, W1_w, W1_b),
              (H2_w, H2_b, W2_w, W2_b),
              (H3_w, H3_b, W3_w, W3_b)]
    fps = []
    for hw, hb, ww, wb in levels:
        al, ah = _sc_aggregate(fl, fh, src, dst)
        fl, fh, fp = _tc_level(gid3, al, ah, hw, hb.reshape(1, D),
                               ww, wb.reshape(1, D))
        fps.append(fp)
    fcb_full = jnp.broadcast_to(fc_b.reshape(1, 1), (G, 1))
    return _fc_out(fps[0], fps[1], fps[2], fc_w, fcb_full)


# speedup vs baseline: 6.4153x; 1.9169x over previous
import functools

import jax
import jax.numpy as jnp
from jax import lax
from jax.experimental import pallas as pl
from jax.experimental.pallas import tpu as pltpu
from jax.experimental.pallas import tpu_sc as plsc

N = 10000
E = 160000
D = 256
DH = 128
G = 64
NS = 16
CHUNK = 80
EPT = E // NS
NCHUNKS = EPT // CHUNK
ROWS_PT = 624
ROWS_REM = N - NS * ROWS_PT

BLK = 1000
NB = N // BLK


NBUF = 2


def _sc_agg_body(fl_hbm, fh_hbm, src3_hbm, dst3_hbm, ol_hbm, oh_hbm,
                 acc, sidx, didx, rows0, rows1, gs0, gs1, ss0, ss1):
    rows = [rows0, rows1]
    gs = [gs0, gs1]
    ss = [ss0, ss1]
    cid = lax.axis_index("c")
    sid = lax.axis_index("s")
    r0 = sid * ROWS_PT
    rem0 = NS * ROWS_PT

    @pl.when(cid == 0)
    def _():
        pltpu.sync_copy(fl_hbm.at[pl.ds(r0, ROWS_PT)], acc.at[pl.ds(r0, ROWS_PT)])

        @pl.when(sid == 0)
        def _():
            pltpu.sync_copy(fl_hbm.at[pl.ds(rem0, ROWS_REM)],
                            acc.at[pl.ds(rem0, ROWS_REM)])

    @pl.when(cid == 1)
    def _():
        pltpu.sync_copy(fh_hbm.at[pl.ds(r0, ROWS_PT)], acc.at[pl.ds(r0, ROWS_PT)])

        @pl.when(sid == 0)
        def _():
            pltpu.sync_copy(fh_hbm.at[pl.ds(rem0, ROWS_REM)],
                            acc.at[pl.ds(rem0, ROWS_REM)])

    pltpu.sync_copy(src3_hbm.at[sid], sidx)
    pltpu.sync_copy(dst3_hbm.at[sid], didx)
    plsc.subcore_barrier()

    def gstart(b, c):
        @pl.when(cid == 0)
        def _():
            pltpu.async_copy(fl_hbm.at[sidx.at[pl.ds(c * CHUNK, CHUNK)]],
                             rows[b], gs[b])

        @pl.when(cid == 1)
        def _():
            pltpu.async_copy(fh_hbm.at[sidx.at[pl.ds(c * CHUNK, CHUNK)]],
                             rows[b], gs[b])

    def gwait(b):
        pltpu.make_async_copy(fl_hbm.at[sidx.at[pl.ds(0, CHUNK)]],
                              rows[b], gs[b]).wait()

    def sstart(b, c):
        pltpu.async_copy(rows[b], acc.at[didx.at[c]], ss[b], add=True)

    def swait(b):
        pltpu.make_async_copy(rows[b], acc.at[didx.at[0]], ss[b]).wait()

    gstart(0, 0)

    def body(j, carry):
        for b in range(NBUF):
            c = NBUF * j + b
            gwait(b)
            sstart(b, c)
            b1 = (b + 1) % NBUF

            @pl.when(c >= 1)
            def _():
                swait(b1)

            @pl.when(c + 1 <= NCHUNKS - 1)
            def _():
                gstart(b1, c + 1)
        return carry

    lax.fori_loop(0, (NCHUNKS - 1) // NBUF, body, 0)
    gwait(0)
    sstart(0, NCHUNKS - 1)
    swait(1)
    swait(0)
    plsc.subcore_barrier()

    @pl.when(cid == 0)
    def _():
        pltpu.sync_copy(acc.at[pl.ds(r0, ROWS_PT)], ol_hbm.at[pl.ds(r0, ROWS_PT)])

        @pl.when(sid == 0)
        def _():
            pltpu.sync_copy(acc.at[pl.ds(rem0, ROWS_REM)],
                            ol_hbm.at[pl.ds(rem0, ROWS_REM)])

    @pl.when(cid == 1)
    def _():
        pltpu.sync_copy(acc.at[pl.ds(r0, ROWS_PT)], oh_hbm.at[pl.ds(r0, ROWS_PT)])

        @pl.when(sid == 0)
        def _():
            pltpu.sync_copy(acc.at[pl.ds(rem0, ROWS_REM)],
                            oh_hbm.at[pl.ds(rem0, ROWS_REM)])


def _sc_aggregate(fl, fh, src, dst):
    mesh = plsc.VectorSubcoreMesh(core_axis_name="c", subcore_axis_name="s")
    f = pl.kernel(
        _sc_agg_body,
        out_type=(jax.ShapeDtypeStruct((N, DH), jnp.float32),
                  jax.ShapeDtypeStruct((N, DH), jnp.float32)),
        mesh=mesh,
        scratch_types=(
            pltpu.VMEM_SHARED((N, DH), jnp.float32),
            pltpu.VMEM((EPT,), jnp.int32),
            pltpu.VMEM((NCHUNKS, CHUNK), jnp.int32),
            pltpu.VMEM((CHUNK, DH), jnp.float32),
            pltpu.VMEM((CHUNK, DH), jnp.float32),
            pltpu.SemaphoreType.DMA,
            pltpu.SemaphoreType.DMA,
            pltpu.SemaphoreType.DMA,
            pltpu.SemaphoreType.DMA,
        ),
        name="sc_edge_aggregate",
    )
    src3 = src.reshape(NS, EPT)
    dst3 = dst.reshape(NS, NCHUNKS, CHUNK)
    return f(fl, fh, src3, dst3)


def _dotT(a, b):
    return lax.dot_general(a, b, (((1,), (1,)), ((), ())),
                           preferred_element_type=jnp.float32)


def _tc_level_body(gid_ref, al_ref, ah_ref, hw_ref, hb_ref, ww_ref, wb_ref,
                   nl_ref, nh_ref, fp_ref):
    step = pl.program_id(0)
    agg = jnp.concatenate([al_ref[...], ah_ref[...]], axis=1)
    nf = jnp.maximum(_dotT(agg, hw_ref[...]) + hb_ref[...], 0.0)
    logits = _dotT(nf, ww_ref[...]) + wb_ref[...]
    m = jnp.max(logits, axis=1, keepdims=True)
    ex = jnp.exp(logits - m)
    sparse = ex / jnp.sum(ex, axis=1, keepdims=True)
    nl_ref[...] = nf[:, :DH]
    nh_ref[...] = nf[:, DH:]
    gid = gid_ref[...].reshape(BLK, 1)
    onehot = (gid == lax.broadcasted_iota(jnp.int32, (BLK, G), 1)).astype(jnp.float32)
    fpb = lax.dot_general(onehot, sparse, (((0,), (0,)), ((), ())),
                          preferred_element_type=jnp.float32,
                          precision=lax.Precision.HIGHEST)

    @pl.when(step == 0)
    def _():
        fp_ref[...] = fpb

    @pl.when(step != 0)
    def _():
        fp_ref[...] += fpb


def _tc_level(gid3, al, ah, hw, hb, ww, wb):
    return pl.pallas_call(
        _tc_level_body,
        grid=(NB,),
        in_specs=[
            pl.BlockSpec((1, 1, BLK), lambda i: (i, 0, 0)),
            pl.BlockSpec((BLK, DH), lambda i: (i, 0)),
            pl.BlockSpec((BLK, DH), lambda i: (i, 0)),
            pl.BlockSpec((D, D), lambda i: (0, 0)),
            pl.BlockSpec((1, D), lambda i: (0, 0)),
            pl.BlockSpec((D, D), lambda i: (0, 0)),
            pl.BlockSpec((1, D), lambda i: (0, 0)),
        ],
        out_specs=[
            pl.BlockSpec((BLK, DH), lambda i: (i, 0)),
            pl.BlockSpec((BLK, DH), lambda i: (i, 0)),
            pl.BlockSpec((G, D), lambda i: (0, 0)),
        ],
        out_shape=[
            jax.ShapeDtypeStruct((N, DH), jnp.float32),
            jax.ShapeDtypeStruct((N, DH), jnp.float32),
            jax.ShapeDtypeStruct((G, D), jnp.float32),
        ],
        name="tc_level_dense",
    )(gid3, al, ah, hw, hb, ww, wb)


def _fc_body(fp1_ref, fp2_ref, fp3_ref, fcw_ref, fcb_ref, out_ref):
    fp = fp1_ref[...] + fp2_ref[...] + fp3_ref[...]
    fpb = fp.astype(jnp.bfloat16).astype(jnp.float32)
    fwb = fcw_ref[...].astype(jnp.bfloat16).astype(jnp.float32)
    prod = fpb * fwb
    out_ref[...] = jnp.sum(prod, axis=1, keepdims=True) + fcb_ref[...]


def _fc_out(fp1, fp2, fp3, fcw, fcb):
    return pl.pallas_call(
        _fc_body,
        out_shape=jax.ShapeDtypeStruct((G, 1), jnp.float32),
        name="tc_fc_out",
    )(fp1, fp2, fp3, fcw, fcb)


def kernel(x, edge_index, graph_ids,
           H1_w, H1_b, W1_w, W1_b,
           H2_w, H2_b, W2_w, W2_b,
           H3_w, H3_b, W3_w, W3_b,
           fc_w, fc_b):
    src = edge_index[0]
    dst = edge_index[1]
    fl = x[:, :DH]
    fh = x[:, DH:]
    gid3 = graph_ids.reshape(NB, 1, BLK)
    levels = [(H1_w, H1_b, W1_w, W1_b),
              (H2_w, H2_b, W2_w, W2_b),
              (H3_w, H3_b, W3_w, W3_b)]
    fps = []
    for hw, hb, ww, wb in levels:
        al, ah = _sc_aggregate(fl, fh, src, dst)
        fl, fh, fp = _tc_level(gid3, al, ah, hw, hb.reshape(1, D),
                               ww, wb.reshape(1, D))
        fps.append(fp)
    fcb_full = jnp.broadcast_to(fc_b.reshape(1, 1), (G, 1))
    return _fc_out(fps[0], fps[1], fps[2], fc_w, fcb_full)
